# baseline (speedup 1.0000x reference)
import jax
import jax.numpy as jnp
from jax import lax
from jax.experimental import pallas as pl
from jax.experimental.pallas import tpu as pltpu

_MAX_MASK_RATIO = 0.1
_G = 128

_MASK_CACHE = {}


def _static_mask_bounds(B, D):
    if (B, D) not in _MASK_CACHE:
        max_mask_len = int(D * _MAX_MASK_RATIO)
        with jax.ensure_compile_time_eval():
            key = jax.random.key(42)
            k1, k2 = jax.random.split(key)
            mask_len = jax.random.randint(k1, (B,), 1, max_mask_len + 1)
            mask_start = jax.random.randint(k2, (B,), 0, D - max_mask_len + 1)
            starts = [int(x) for x in mask_start]
            ends = [int(s + l) for s, l in zip(starts, [int(x) for x in mask_len])]
        _MASK_CACHE[(B, D)] = list(zip(starts, ends))
    return _MASK_CACHE[(B, D)]


def kernel(mean):
    B, T, D = mean.shape
    bounds = _static_mask_bounds(B, D)

    plans = []
    for b, (s, e) in enumerate(bounds):
        glo, ghi = s // _G, (e - 1) // _G
        if glo * _G > 0:
            plans.append((b, "copy", 0, glo * _G, s, e))
        for g in range(glo, ghi + 1):
            c0, c1 = g * _G, (g + 1) * _G
            if s <= c0 and e >= c1:
                plans.append((b, "zero", c0, c1, s, e))
            else:
                plans.append((b, "edge", c0, c1, s, e))
        if (ghi + 1) * _G < D:
            plans.append((b, "copy", (ghi + 1) * _G, D, s, e))

    n_edge = sum(1 for p in plans if p[1] == "edge")

    def body(x_hbm, o_hbm, zbuf, ebufs, zsem, csem, esem_in, esem_out):
        zbuf[...] = jnp.zeros((T, _G), jnp.float32)
        copies = []
        for b, kind, c0, c1, s, e in plans:
            if kind == "copy":
                copies.append(pltpu.make_async_copy(
                    x_hbm.at[b, :, pl.ds(c0, c1 - c0)],
                    o_hbm.at[b, :, pl.ds(c0, c1 - c0)], csem))
                copies[-1].start()
            elif kind == "zero":
                copies.append(pltpu.make_async_copy(
                    zbuf, o_hbm.at[b, :, pl.ds(c0, _G)], zsem))
                copies[-1].start()
        edges = [p for p in plans if p[1] == "edge"]
        ins = []
        for i, (b, kind, c0, c1, s, e) in enumerate(edges):
            d = pltpu.make_async_copy(
                x_hbm.at[b, :, pl.ds(c0, _G)], ebufs.at[i], esem_in)
            d.start()
            ins.append(d)
        outs = []
        for i, (b, kind, c0, c1, s, e) in enumerate(edges):
            ins[i].wait()
            col = c0 + lax.broadcasted_iota(jnp.int32, (T, _G), 1)
            keep = (col < s) | (col >= e)
            ebufs[i] = jnp.where(keep, ebufs[i], 0.0)
            d = pltpu.make_async_copy(
                ebufs.at[i], o_hbm.at[b, :, pl.ds(c0, _G)], esem_out)
            d.start()
            outs.append(d)
        for d in copies:
            d.wait()
        for d in outs:
            d.wait()

    return pl.pallas_call(
        body,
        in_specs=[pl.BlockSpec(memory_space=pl.ANY)],
        out_specs=pl.BlockSpec(memory_space=pl.ANY),
        out_shape=jax.ShapeDtypeStruct((B, T, D), mean.dtype),
        scratch_shapes=[
            pltpu.VMEM((T, _G), jnp.float32),
            pltpu.VMEM((n_edge, T, _G), jnp.float32),
            pltpu.SemaphoreType.DMA,
            pltpu.SemaphoreType.DMA,
            pltpu.SemaphoreType.DMA,
            pltpu.SemaphoreType.DMA,
        ],
    )(mean)


# speedup vs baseline: 31.8403x; 31.8403x over previous
import functools

import jax
import jax.numpy as jnp
from jax import lax
from jax.experimental import pallas as pl
from jax.experimental.pallas import tpu as pltpu
from jax.experimental.pallas import tpu_sc as plsc

_MAX_MASK_RATIO = 0.1
_LANES = 16
_CH = 16
_NBUF = 3

_MASK_CACHE = {}


def _static_mask_bounds(B, D):
    if (B, D) not in _MASK_CACHE:
        max_mask_len = int(D * _MAX_MASK_RATIO)
        with jax.ensure_compile_time_eval():
            key = jax.random.key(42)
            k1, k2 = jax.random.split(key)
            mask_len = jax.random.randint(k1, (B,), 1, max_mask_len + 1)
            mask_start = jax.random.randint(k2, (B,), 0, D - max_mask_len + 1)
            starts = [int(x) for x in mask_start]
            ends = [int(s + l) for s, l in zip(starts, [int(x) for x in mask_len])]
        _MASK_CACHE[(B, D)] = list(zip(starts, ends))
    return _MASK_CACHE[(B, D)]


def kernel(mean):
    B, T, D = mean.shape
    bounds = _static_mask_bounds(B, D)
    num_cores, num_subcores = 2, 16
    NW = num_cores * num_subcores
    rows_per_worker = T // NW
    nchunks_total = B * (rows_per_worker // _CH)
    nchunks_batch = rows_per_worker // _CH
    mesh = plsc.VectorSubcoreMesh(
        core_axis_name="c", subcore_axis_name="s",
        num_cores=num_cores, num_subcores=num_subcores)

    @functools.partial(
        pl.kernel,
        out_type=jax.ShapeDtypeStruct((B, T, D), mean.dtype),
        mesh=mesh,
        scratch_types=[
            pltpu.VMEM((_CH, D), jnp.float32),
            pltpu.VMEM((_CH, D), jnp.float32),
            pltpu.VMEM((_CH, D), jnp.float32),
            pltpu.SemaphoreType.DMA,
            pltpu.SemaphoreType.DMA,
            pltpu.SemaphoreType.DMA,
            pltpu.SemaphoreType.DMA,
            pltpu.SemaphoreType.DMA,
            pltpu.SemaphoreType.DMA,
        ],
    )
    def sc_kernel(mean_hbm, out_hbm, buf0, buf1, buf2,
                  isem0, isem1, isem2, osem0, osem1, osem2):
        wid = lax.axis_index("s") * num_cores + lax.axis_index("c")
        r0 = wid * rows_per_worker
        lane = lax.broadcasted_iota(jnp.int32, (_LANES,), 0)
        bufs = (buf0, buf1, buf2)
        isems = (isem0, isem1, isem2)
        osems = (osem0, osem1, osem2)

        s_consts = [jnp.int32(s) for s, _ in bounds]
        e_consts = [jnp.int32(e) for _, e in bounds]

        def chunk_refs(j):
            b = j // nchunks_batch
            row = r0 + (j - b * nchunks_batch) * _CH
            return b, row

        def start_in(j, k):
            b, row = chunk_refs(j)
            return pltpu.async_copy(
                mean_hbm.at[b, pl.ds(row, _CH), :], bufs[k], isems[k])

        def start_out(j, k):
            b, row = chunk_refs(j)
            return pltpu.async_copy(
                bufs[k], out_hbm.at[b, pl.ds(row, _CH), :], osems[k])

        def wait_out(k):
            pltpu.make_async_copy(
                bufs[k], out_hbm.at[0, pl.ds(r0, _CH), :], osems[k]).wait()

        def zero_stripe(k, j):
            b = j // nchunks_batch
            s = s_consts[-1]
            e = e_consts[-1]
            for bb in range(B - 1):
                s = jnp.where(b == bb, s_consts[bb], s)
                e = jnp.where(b == bb, e_consts[bb], e)
            g_lo = s // _LANES
            g_hi = (e + _LANES - 1) // _LANES

            def g_fn(g, carry):
                base = pl.multiple_of(g * _LANES, _LANES)
                col = base + lane
                keep = (col < s) | (col >= e)
                for r in range(_CH):
                    x = bufs[k][r, pl.ds(base, _LANES)]
                    bufs[k][r, pl.ds(base, _LANES)] = jnp.where(keep, x, 0.0)
                return carry
            lax.fori_loop(g_lo, g_hi, g_fn, 0)

        def process(j, k):
            jn = j + _NBUF - 1
            kn = (k + _NBUF - 1) % _NBUF

            @pl.when(jn < nchunks_total)
            def _():
                @pl.when(jn >= _NBUF)
                def _():
                    wait_out(kn)
                start_in(jn, kn)

            pltpu.make_async_copy(
                mean_hbm.at[0, pl.ds(r0, _CH), :], bufs[k], isems[k]).wait()
            zero_stripe(k, j)
            start_out(j, k)

        start_in(jnp.int32(0), 0)
        start_in(jnp.int32(1), 1)

        def loop_body(q, carry):
            j = q * _NBUF
            for k in range(_NBUF):
                process(j + k, k)
            return carry
        n_full = nchunks_total // _NBUF
        lax.fori_loop(0, n_full, loop_body, 0)
        for jj in range(n_full * _NBUF, nchunks_total):
            process(jnp.int32(jj), jj % _NBUF)
        for k in range(min(_NBUF, nchunks_total)):
            wait_out(k)

    return sc_kernel(mean)


# speedup vs baseline: 41.5172x; 1.3039x over previous
import jax
import jax.numpy as jnp
from jax import lax
from jax.experimental import pallas as pl
from jax.experimental.pallas import tpu as pltpu

_MAX_MASK_RATIO = 0.1
_T_BLK = 512

_MASK_CACHE = {}


def _static_mask_bounds(B, D):
    if (B, D) not in _MASK_CACHE:
        max_mask_len = int(D * _MAX_MASK_RATIO)
        with jax.ensure_compile_time_eval():
            key = jax.random.key(42)
            k1, k2 = jax.random.split(key)
            mask_len = jax.random.randint(k1, (B,), 1, max_mask_len + 1)
            mask_start = jax.random.randint(k2, (B,), 0, D - max_mask_len + 1)
            starts = [int(x) for x in mask_start]
            ends = [int(s + l) for s, l in zip(starts, [int(x) for x in mask_len])]
        _MASK_CACHE[(B, D)] = list(zip(starts, ends))
    return _MASK_CACHE[(B, D)]


def kernel(mean):
    B, T, D = mean.shape
    bounds = _static_mask_bounds(B, D)
    starts = jnp.array([s for s, _ in bounds], dtype=jnp.int32)
    ends = jnp.array([e for _, e in bounds], dtype=jnp.int32)

    def body(starts_ref, ends_ref, x_ref, o_ref):
        b = pl.program_id(0)
        s = starts_ref[b]
        e = ends_ref[b]
        col = lax.broadcasted_iota(jnp.int32, (_T_BLK, D), 1)
        keep = (col < s) | (col >= e)
        o_ref[0] = jnp.where(keep, x_ref[0], 0.0)

    return pl.pallas_call(
        body,
        grid=(B, T // _T_BLK),
        in_specs=[
            pl.BlockSpec(memory_space=pltpu.SMEM),
            pl.BlockSpec(memory_space=pltpu.SMEM),
            pl.BlockSpec((1, _T_BLK, D), lambda b, t: (b, t, 0)),
        ],
        out_specs=pl.BlockSpec((1, _T_BLK, D), lambda b, t: (b, t, 0)),
        out_shape=jax.ShapeDtypeStruct((B, T, D), mean.dtype),
    )(starts, ends, mean)


# speedup vs baseline: 42.2743x; 1.0182x over previous
import jax
import jax.numpy as jnp
from jax import lax
from jax.experimental import pallas as pl
from jax.experimental.pallas import tpu as pltpu

_MAX_MASK_RATIO = 0.1
_T_BLK = 1024

_MASK_CACHE = {}


def _static_mask_bounds(B, D):
    if (B, D) not in _MASK_CACHE:
        max_mask_len = int(D * _MAX_MASK_RATIO)
        with jax.ensure_compile_time_eval():
            key = jax.random.key(42)
            k1, k2 = jax.random.split(key)
            mask_len = jax.random.randint(k1, (B,), 1, max_mask_len + 1)
            mask_start = jax.random.randint(k2, (B,), 0, D - max_mask_len + 1)
            starts = [int(x) for x in mask_start]
            ends = [int(s + l) for s, l in zip(starts, [int(x) for x in mask_len])]
        _MASK_CACHE[(B, D)] = list(zip(starts, ends))
    return _MASK_CACHE[(B, D)]


def kernel(mean):
    B, T, D = mean.shape
    bounds = _static_mask_bounds(B, D)
    starts = jnp.array([s for s, _ in bounds], dtype=jnp.int32)
    ends = jnp.array([e for _, e in bounds], dtype=jnp.int32)

    def body(starts_ref, ends_ref, x_ref, o_ref):
        b = pl.program_id(0)
        s = starts_ref[b]
        e = ends_ref[b]
        col = lax.broadcasted_iota(jnp.int32, (_T_BLK, D), 1)
        keep = (col < s) | (col >= e)
        o_ref[0] = jnp.where(keep, x_ref[0], 0.0)

    return pl.pallas_call(
        body,
        grid=(B, T // _T_BLK),
        in_specs=[
            pl.BlockSpec(memory_space=pltpu.SMEM),
            pl.BlockSpec(memory_space=pltpu.SMEM),
            pl.BlockSpec((1, _T_BLK, D), lambda b, t: (b, t, 0)),
        ],
        out_specs=pl.BlockSpec((1, _T_BLK, D), lambda b, t: (b, t, 0)),
        out_shape=jax.ShapeDtypeStruct((B, T, D), mean.dtype),
    )(starts, ends, mean)
